# baseline (speedup 1.0000x reference)
import dataclasses
import functools

import jax
import jax.numpy as jnp
from jax.experimental import pallas as pl
from jax.experimental.pallas import tpu as pltpu
from jax.experimental.pallas import tpu_sc as plsc

_T, _D, _H, _E, _K = 64, 1024, 2048, 64, 2
_HC = 2048
_L = 16
_NCH = _E // _L


def _router_tc_kernel(x_ref, Wr_ref, br_ref, Wn_ref, bn_ref, noise_ref,
                      G_ref, am_ref):
    x = x_ref[...]
    logits = jnp.dot(x, Wr_ref[...], preferred_element_type=jnp.float32) + br_ref[...]
    nl = jnp.dot(x, Wn_ref[...], preferred_element_type=jnp.float32) + bn_ref[...]
    noisy = logits + noise_ref[...] * jax.nn.softplus(nl)

    ecols = jax.lax.broadcasted_iota(jnp.int32, (_T, _E), 1)
    m0 = jnp.max(noisy, axis=1, keepdims=True)
    i0 = jnp.min(jnp.where(noisy == m0, ecols, _E), axis=1, keepdims=True)
    masked = jnp.where(ecols == i0, -jnp.inf, noisy)
    m1 = jnp.max(masked, axis=1, keepdims=True)
    i1 = jnp.min(jnp.where(masked == m1, ecols, _E), axis=1, keepdims=True)
    r = jnp.exp(m1 - m0)
    g0 = 1.0 / (1.0 + r)
    g1 = r / (1.0 + r)
    G_ref[...] = jnp.where(ecols == i0, g0, 0.0) + jnp.where(ecols == i1, g1, 0.0)
    sel = ((ecols == i0) | (ecols == i1)).astype(jnp.int32)
    am_ref[...] = jnp.max(sel, axis=0, keepdims=True)


def _sc_compact_kernel(am_hbm, ids_hbm, n_hbm, am_s, ids_s, n16_s, sem):
    core = jax.lax.axis_index("c")

    @pl.when(core == 0)
    def _compact():
        pltpu.async_copy(am_hbm, am_s, sem).wait()
        n16_s[0] = 0

        @pl.loop(0, _E)
        def _scan(e):
            @pl.when(am_s[e] > 0)
            def _take():
                ids_s[n16_s[0]] = e
                n16_s[0] = n16_s[0] + 1

        n = n16_s[0]
        last = ids_s[n - 1]

        @pl.loop(0, _E)
        def _pad(j):
            @pl.when(j >= n)
            def _fill():
                ids_s[j] = last

        @pl.loop(1, _L)
        def _bn(i):
            n16_s[i] = n

        pltpu.async_copy(ids_s, ids_hbm, sem).wait()
        pltpu.async_copy(n16_s, n_hbm, sem).wait()


def _make_sc_compact():
    return functools.partial(
        pl.kernel,
        out_type=[
            jax.ShapeDtypeStruct((_E,), jnp.int32),
            jax.ShapeDtypeStruct((_L,), jnp.int32),
        ],
        mesh=plsc.ScalarSubcoreMesh(axis_name="c", num_cores=2),
        scratch_types=[
            pltpu.SMEM((_E,), jnp.int32),
            pltpu.SMEM((_E,), jnp.int32),
            pltpu.SMEM((_L,), jnp.int32),
            pltpu.SemaphoreType.DMA,
        ],
    )(_sc_compact_kernel)


def _ffn_kernel(ids_ref, n_ref, x_ref, G_ref, w1_ref, w3_ref, w2_ref, out_ref):
    j = pl.program_id(0)

    @pl.when(j == 0)
    def _init():
        out_ref[...] = jnp.zeros_like(out_ref)

    @pl.when(j < n_ref[0])
    def _body():
        xb = x_ref[...].astype(jnp.bfloat16)
        hp = jnp.dot(xb, w1_ref[0].astype(jnp.bfloat16),
                     preferred_element_type=jnp.float32)
        gp = jnp.dot(xb, w3_ref[0].astype(jnp.bfloat16),
                     preferred_element_type=jnp.float32)
        s = (hp * jax.nn.sigmoid(hp) * gp).astype(jnp.bfloat16)
        y = jnp.dot(s, w2_ref[0].astype(jnp.bfloat16),
                    preferred_element_type=jnp.float32)
        e = ids_ref[j]
        ecols = jax.lax.broadcasted_iota(jnp.int32, (_T, _E), 1)
        gcol = jnp.sum(jnp.where(ecols == e, G_ref[...], 0.0),
                       axis=1, keepdims=True)
        out_ref[...] += y * gcol


def kernel(x, Wr, br, Wn, bn, w1, w2, w3):
    noise = jax.random.normal(jax.random.key(1234), (_T, _E), dtype=jnp.float32)
    G, am2d = pl.pallas_call(
        _router_tc_kernel,
        out_shape=[
            jax.ShapeDtypeStruct((_T, _E), jnp.float32),
            jax.ShapeDtypeStruct((1, _E), jnp.int32),
        ],
    )(x, Wr, br.reshape(1, _E), Wn, bn.reshape(1, _E), noise)

    ids, n16 = _make_sc_compact()(am2d.reshape(_E))
    n = n16[0:1]

    out = pl.pallas_call(
        _ffn_kernel,
        grid_spec=pltpu.PrefetchScalarGridSpec(
            num_scalar_prefetch=2,
            grid=(_E,),
            in_specs=[
                pl.BlockSpec((_T, _D), lambda j, ids, n: (0, 0)),
                pl.BlockSpec((_T, _E), lambda j, ids, n: (0, 0)),
                pl.BlockSpec((1, _D, _HC), lambda j, ids, n: (ids[j], 0, 0)),
                pl.BlockSpec((1, _D, _HC), lambda j, ids, n: (ids[j], 0, 0)),
                pl.BlockSpec((1, _HC, _D), lambda j, ids, n: (ids[j], 0, 0)),
            ],
            out_specs=pl.BlockSpec((_T, _D), lambda j, ids, n: (0, 0)),
        ),
        out_shape=jax.ShapeDtypeStruct((_T, _D), jnp.float32),
        compiler_params=pltpu.CompilerParams(
            dimension_semantics=("arbitrary",),
        ),
    )(ids, n, x, G, w1, w3, w2)
    return out


# speedup vs baseline: 1.0026x; 1.0026x over previous
import functools

import jax
import jax.numpy as jnp
from jax.experimental import pallas as pl
from jax.experimental.pallas import tpu as pltpu
from jax.experimental.pallas import tpu_sc as plsc

_T, _D, _H, _E, _K = 64, 1024, 2048, 64, 2
_HC = 2048
_L = 16


def _router_tc_kernel(x_ref, Wr_ref, br_ref, Wn_ref, bn_ref, noise_ref,
                      G_ref, am_ref):
    x = x_ref[...]
    logits = jnp.dot(x, Wr_ref[...], preferred_element_type=jnp.float32) + br_ref[...]
    nl = jnp.dot(x, Wn_ref[...], preferred_element_type=jnp.float32) + bn_ref[...]
    noisy = logits + noise_ref[...] * jax.nn.softplus(nl)

    ecols = jax.lax.broadcasted_iota(jnp.int32, (_T, _E), 1)
    m0 = jnp.max(noisy, axis=1, keepdims=True)
    i0 = jnp.min(jnp.where(noisy == m0, ecols, _E), axis=1, keepdims=True)
    masked = jnp.where(ecols == i0, -jnp.inf, noisy)
    m1 = jnp.max(masked, axis=1, keepdims=True)
    i1 = jnp.min(jnp.where(masked == m1, ecols, _E), axis=1, keepdims=True)
    r = jnp.exp(m1 - m0)
    g0 = 1.0 / (1.0 + r)
    g1 = r / (1.0 + r)
    G_ref[...] = jnp.where(ecols == i0, g0, 0.0) + jnp.where(ecols == i1, g1, 0.0)
    sel = ((ecols == i0) | (ecols == i1)).astype(jnp.int32)
    am_ref[...] = jnp.max(sel, axis=0, keepdims=True)


def _sc_compact_kernel(am_hbm, ids_hbm, n_hbm, am_s, ids_s, n16_s, sem):
    core = jax.lax.axis_index("c")

    @pl.when(core == 0)
    def _compact():
        pltpu.async_copy(am_hbm, am_s, sem).wait()
        n16_s[0] = 0

        @pl.loop(0, _E)
        def _scan(e):
            @pl.when(am_s[e] > 0)
            def _take():
                ids_s[n16_s[0]] = e
                n16_s[0] = n16_s[0] + 1

        n = n16_s[0]
        last = ids_s[n - 1]

        @pl.loop(0, _E)
        def _pad(j):
            @pl.when(j >= n)
            def _fill():
                ids_s[j] = last

        @pl.loop(1, _L)
        def _bn(i):
            n16_s[i] = n

        pltpu.async_copy(ids_s, ids_hbm, sem).wait()
        pltpu.async_copy(n16_s, n_hbm, sem).wait()


def _make_sc_compact():
    return functools.partial(
        pl.kernel,
        out_type=[
            jax.ShapeDtypeStruct((_E,), jnp.int32),
            jax.ShapeDtypeStruct((_L,), jnp.int32),
        ],
        mesh=plsc.ScalarSubcoreMesh(axis_name="c", num_cores=2),
        scratch_types=[
            pltpu.SMEM((_E,), jnp.int32),
            pltpu.SMEM((_E,), jnp.int32),
            pltpu.SMEM((_L,), jnp.int32),
            pltpu.SemaphoreType.DMA,
        ],
    )(_sc_compact_kernel)


def _ffn_kernel(ids_ref, n_ref, x_ref, G_ref, w1_ref, w3_ref, w2_ref, out_ref):
    j = pl.program_id(0)

    @pl.when(j == 0)
    def _init():
        out_ref[...] = jnp.zeros_like(out_ref)

    @pl.when(j < n_ref[0])
    def _body():
        xb = x_ref[...].astype(jnp.bfloat16)
        hp = jnp.dot(xb, w1_ref[0].astype(jnp.bfloat16),
                     preferred_element_type=jnp.float32)
        gp = jnp.dot(xb, w3_ref[0].astype(jnp.bfloat16),
                     preferred_element_type=jnp.float32)
        s = (hp * jax.nn.sigmoid(hp) * gp).astype(jnp.bfloat16)
        y = jnp.dot(s, w2_ref[0].astype(jnp.bfloat16),
                    preferred_element_type=jnp.float32)
        e = ids_ref[j]
        ecols = jax.lax.broadcasted_iota(jnp.int32, (_T, _E), 1)
        gcol = jnp.sum(jnp.where(ecols == e, G_ref[...], 0.0),
                       axis=1, keepdims=True)
        out_ref[...] += y * gcol


def kernel(x, Wr, br, Wn, bn, w1, w2, w3):
    noise = jax.random.normal(jax.random.key(1234), (_T, _E), dtype=jnp.float32)
    G, am2d = pl.pallas_call(
        _router_tc_kernel,
        out_shape=[
            jax.ShapeDtypeStruct((_T, _E), jnp.float32),
            jax.ShapeDtypeStruct((1, _E), jnp.int32),
        ],
    )(x, Wr, br.reshape(1, _E), Wn, bn.reshape(1, _E), noise)

    ids, n16 = _make_sc_compact()(am2d.reshape(_E))
    n = n16[0:1]

    out = pl.pallas_call(
        _ffn_kernel,
        grid_spec=pltpu.PrefetchScalarGridSpec(
            num_scalar_prefetch=2,
            grid=(_E,),
            in_specs=[
                pl.BlockSpec((_T, _D), lambda j, ids, n: (0, 0)),
                pl.BlockSpec((_T, _E), lambda j, ids, n: (0, 0)),
                pl.BlockSpec((1, _D, _HC), lambda j, ids, n: (ids[j], 0, 0)),
                pl.BlockSpec((1, _D, _HC), lambda j, ids, n: (ids[j], 0, 0)),
                pl.BlockSpec((1, _HC, _D), lambda j, ids, n: (ids[j], 0, 0)),
            ],
            out_specs=pl.BlockSpec((_T, _D), lambda j, ids, n: (0, 0)),
        ),
        out_shape=jax.ShapeDtypeStruct((_T, _D), jnp.float32),
        compiler_params=pltpu.CompilerParams(
            dimension_semantics=("arbitrary",),
        ),
    )(ids, n, x, G, w1, w3, w2)
    return out
